# baseline (speedup 1.0000x reference)
import functools

import jax
import jax.numpy as jnp
from jax import lax
from jax.experimental import pallas as pl
from jax.experimental.pallas import tpu as pltpu
from jax.experimental.pallas import tpu_sc as plsc

N = 10000
E = 320000
D = 128
NR = 4
NUM_VOCAB = 5000
MAX_SEQ_LEN = 5
NUM_GRAPHS = 128
NUM_NODE_TYPES = 100
NUM_NODE_ATTRS = 1000
MAX_DEPTH = 20
LAYER_TIMESTEPS = [2, 2, 1, 2, 1]
RESIDUALS_MAP = {2: [0], 4: [0, 2]}

NTILES = 32
K = 112
C0 = 72
C1 = 108
T0 = K * C0
T1 = K * C1
EPAD = 16 * (T0 + T1)
NACC = 10112
ZR = NACC // 16

RB = 1000
NBLK = N // RB
RBE = 200
NBLKE = N // RBE


_sc_mesh = plsc.VectorSubcoreMesh(
    core_axis_name="c", subcore_axis_name="s", num_cores=2, num_subcores=16)


@functools.partial(
    pl.kernel,
    mesh=_sc_mesh,
    out_type=jax.ShapeDtypeStruct((2, NACC, D), jnp.float32),
    scratch_types=[
        pltpu.VMEM((K,), jnp.int32),
        pltpu.VMEM((K,), jnp.int32),
        pltpu.VMEM((K,), jnp.int32),
        pltpu.VMEM((K,), jnp.int32),
        pltpu.VMEM((K, D), jnp.float32),
        pltpu.VMEM((K, D), jnp.float32),
        pltpu.VMEM_SHARED((NACC, D), jnp.float32),
        pltpu.SemaphoreType.DMA,
        pltpu.SemaphoreType.DMA,
        pltpu.SemaphoreType.DMA,
        pltpu.SemaphoreType.DMA,
    ],
)
def _edge_pass(hr_hbm, gidx_hbm, dst_hbm, zeros_hbm, agg_hbm,
               g0_v, d0_v, g1_v, d1_v, rows0_v, rows1_v,
               acc_sh, semg0, semg1, semi0, semi1):
    cid = lax.axis_index("c")
    sid = lax.axis_index("s")
    nchunks = lax.select(cid == 0, C0, C1)
    base = cid * (16 * T0) + sid * lax.select(cid == 0, T0, T1)
    pltpu.sync_copy(zeros_hbm, acc_sh.at[pl.ds(sid * ZR, ZR)])
    plsc.subcore_barrier()

    def idxcpy(c, gbuf, dbuf, sem):
        off = base + c * K
        pltpu.async_copy(gidx_hbm.at[pl.ds(off, K)], gbuf, sem)
        pltpu.async_copy(dst_hbm.at[pl.ds(off, K)], dbuf, sem)

    def idxwait(c, gbuf, dbuf, sem):
        off = base + lax.select(c < nchunks, c, 0) * K
        pltpu.make_async_copy(gidx_hbm.at[pl.ds(off, K)], gbuf, sem).wait()
        pltpu.make_async_copy(dst_hbm.at[pl.ds(off, K)], dbuf, sem).wait()

    idxcpy(0, g0_v, d0_v, semi0)
    idxwait(0, g0_v, d0_v, semi0)
    pltpu.async_copy(hr_hbm.at[g0_v], rows0_v, semg0)
    idxcpy(1, g1_v, d1_v, semi1)

    def halfstep(c, gY, dY, rowsY, semgY, semiY,
                 gX, dX, rowsX, semgX, semiX):
        idxwait(c + 1, gY, dY, semiY)
        pltpu.async_copy(hr_hbm.at[gY], rowsY, semgY)
        pltpu.make_async_copy(hr_hbm.at[gX], rowsX, semgX).wait()
        pltpu.sync_copy(rowsX, acc_sh.at[dX], add=True)
        nxt = lax.select(c + 2 < nchunks, c + 2, 0)
        idxcpy(nxt, gX, dX, semiX)

    def body(j, carry):
        c0 = 2 * j
        halfstep(c0, g1_v, d1_v, rows1_v, semg1, semi1,
                 g0_v, d0_v, rows0_v, semg0, semi0)
        halfstep(c0 + 1, g0_v, d0_v, rows0_v, semg0, semi0,
                 g1_v, d1_v, rows1_v, semg1, semi1)
        return carry

    lax.fori_loop(0, nchunks // 2, body, 0)
    pltpu.make_async_copy(hr_hbm.at[g0_v], rows0_v, semg0).wait()
    idxwait(0, g1_v, d1_v, semi1)
    plsc.subcore_barrier()
    pltpu.sync_copy(acc_sh.at[pl.ds(sid * ZR, ZR)],
                    agg_hbm.at[cid, pl.ds(sid * ZR, ZR)])


def _hr_body(h_ref, w_ref, out_ref):
    out_ref[0] = jnp.dot(h_ref[...], w_ref[0],
                         preferred_element_type=jnp.float32)


_hr_call = pl.pallas_call(
    _hr_body,
    grid=(NR, NBLK),
    in_specs=[
        pl.BlockSpec((RB, D), lambda r, i: (i, 0)),
        pl.BlockSpec((1, D, D), lambda r, i: (r, 0, 0)),
    ],
    out_specs=pl.BlockSpec((1, RB, D), lambda r, i: (r, i, 0)),
    out_shape=jax.ShapeDtypeStruct((NR, N, D), jnp.float32),
)


def _make_gru(nres):
    def body(*refs):
        a_ref, h_ref, wiaT_ref, whhT_ref, bih_ref, bhh_ref = refs[:6]
        res_refs = refs[6:6 + 2 * nres]
        out_ref = refs[6 + 2 * nres]
        agg = a_ref[0] + a_ref[1]
        gi = jnp.dot(agg, wiaT_ref[...],
                     preferred_element_type=jnp.float32) + bih_ref[...]
        for j in range(nres):
            gi = gi + jnp.dot(res_refs[2 * j][...], res_refs[2 * j + 1][...],
                              preferred_element_type=jnp.float32)
        h = h_ref[...]
        gh = jnp.dot(h, whhT_ref[...],
                     preferred_element_type=jnp.float32) + bhh_ref[...]
        r = jax.nn.sigmoid(gi[:, :D] + gh[:, :D])
        z = jax.nn.sigmoid(gi[:, D:2 * D] + gh[:, D:2 * D])
        n = jnp.tanh(gi[:, 2 * D:] + r * gh[:, 2 * D:])
        out_ref[...] = (1.0 - z) * n + z * h

    in_specs = [
        pl.BlockSpec((2, RB, D), lambda i: (0, i, 0)),
        pl.BlockSpec((RB, D), lambda i: (i, 0)),
        pl.BlockSpec((D, 3 * D), lambda i: (0, 0)),
        pl.BlockSpec((D, 3 * D), lambda i: (0, 0)),
        pl.BlockSpec((1, 3 * D), lambda i: (0, 0)),
        pl.BlockSpec((1, 3 * D), lambda i: (0, 0)),
    ]
    for _ in range(nres):
        in_specs.append(pl.BlockSpec((RB, D), lambda i: (i, 0)))
        in_specs.append(pl.BlockSpec((D, 3 * D), lambda i: (0, 0)))
    return pl.pallas_call(
        body,
        grid=(NBLK,),
        in_specs=in_specs,
        out_specs=pl.BlockSpec((RB, D), lambda i: (i, 0)),
        out_shape=jax.ShapeDtypeStruct((N, D), jnp.float32),
    )


_gru_calls = {nres: _make_gru(nres) for nres in (0, 1, 2)}


def _enc_body(x_ref, te_ref, ae_ref, de_ref, out_ref):
    xin = x_ref[...]
    t = xin[:, 0:1]
    oh = (t == lax.broadcasted_iota(jnp.int32, (RBE, NUM_NODE_TYPES), 1))
    h = jnp.dot(oh.astype(jnp.float32), te_ref[...],
                preferred_element_type=jnp.float32,
                precision=lax.Precision.HIGHEST)
    d = xin[:, 2:3]
    ohd = (d == lax.broadcasted_iota(jnp.int32, (RBE, MAX_DEPTH), 1))
    h = h + jnp.dot(ohd.astype(jnp.float32), de_ref[...],
                    preferred_element_type=jnp.float32,
                    precision=lax.Precision.HIGHEST)
    a = xin[:, 1:2]
    for c in range(8):
        ids = lax.broadcasted_iota(jnp.int32, (RBE, 128), 1) + c * 128
        ohc = (a == ids).astype(jnp.float32)
        h = h + jnp.dot(ohc, ae_ref[c * 128:(c + 1) * 128, :],
                        preferred_element_type=jnp.float32,
                        precision=lax.Precision.HIGHEST)
    out_ref[...] = h


_enc_call = pl.pallas_call(
    _enc_body,
    grid=(NBLKE,),
    in_specs=[
        pl.BlockSpec((RBE, 128), lambda i: (i, 0)),
        pl.BlockSpec((NUM_NODE_TYPES, D), lambda i: (0, 0)),
        pl.BlockSpec((1024, D), lambda i: (0, 0)),
        pl.BlockSpec((MAX_DEPTH, D), lambda i: (0, 0)),
    ],
    out_specs=pl.BlockSpec((RBE, D), lambda i: (i, 0)),
    out_shape=jax.ShapeDtypeStruct((N, D), jnp.float32),
)


def _cls_body(h_ref, h0_ref, clw1_ref, clw2_ref, crw1_ref, crw2_ref,
              clb_ref, crb_ref, b_ref, out_ref):
    h = h_ref[...]
    h0 = h0_ref[...]
    t1 = (jnp.dot(h, clw1_ref[...], preferred_element_type=jnp.float32)
          + jnp.dot(h0, clw2_ref[...], preferred_element_type=jnp.float32)
          + clb_ref[...])
    t2 = (jnp.dot(h, crw1_ref[...], preferred_element_type=jnp.float32)
          + jnp.dot(h0, crw2_ref[...], preferred_element_type=jnp.float32)
          + crb_ref[...])
    node_out = jax.nn.sigmoid(t1) * jnp.tanh(t2)
    b = b_ref[0]
    oh = (b == lax.broadcasted_iota(jnp.int32, (NUM_GRAPHS, RBE), 0))

    @pl.when(pl.program_id(0) == 0)
    def _():
        out_ref[...] = jnp.zeros_like(out_ref)

    out_ref[...] += jnp.dot(oh.astype(jnp.float32), node_out,
                            preferred_element_type=jnp.float32,
                            precision=lax.Precision.HIGHEST)


_cls_call = pl.pallas_call(
    _cls_body,
    grid=(NBLKE,),
    in_specs=[
        pl.BlockSpec((RBE, D), lambda i: (i, 0)),
        pl.BlockSpec((RBE, D), lambda i: (i, 0)),
        pl.BlockSpec((D, D), lambda i: (0, 0)),
        pl.BlockSpec((D, D), lambda i: (0, 0)),
        pl.BlockSpec((D, D), lambda i: (0, 0)),
        pl.BlockSpec((D, D), lambda i: (0, 0)),
        pl.BlockSpec((1, D), lambda i: (0, 0)),
        pl.BlockSpec((1, D), lambda i: (0, 0)),
        pl.BlockSpec((1, 1, RBE), lambda i: (i, 0, 0)),
    ],
    out_specs=pl.BlockSpec((NUM_GRAPHS, D), lambda i: (0, 0)),
    out_shape=jax.ShapeDtypeStruct((NUM_GRAPHS, D), jnp.float32),
)

def _pred_body(g_ref, pw_ref, pb_ref, out_ref):
    out_ref[0] = (jnp.dot(g_ref[...], pw_ref[0],
                          preferred_element_type=jnp.float32) + pb_ref[0])


_pred_call = pl.pallas_call(
    _pred_body,
    grid=(MAX_SEQ_LEN,),
    in_specs=[
        pl.BlockSpec((NUM_GRAPHS, D), lambda s: (0, 0)),
        pl.BlockSpec((1, D, NUM_VOCAB), lambda s: (s, 0, 0)),
        pl.BlockSpec((1, 1, NUM_VOCAB), lambda s: (s, 0, 0)),
    ],
    out_specs=pl.BlockSpec((1, NUM_GRAPHS, NUM_VOCAB), lambda s: (s, 0, 0)),
    out_shape=jax.ShapeDtypeStruct((MAX_SEQ_LEN, NUM_GRAPHS, NUM_VOCAB),
                                   jnp.float32),
)


def kernel(x, edge_index, node_depth, batch, edge_attr, params):
    x = x.astype(jnp.int32)
    src = edge_index[0].astype(jnp.int32)
    dst = edge_index[1].astype(jnp.int32)
    et = edge_attr.astype(jnp.int32)

    gidx_p = jnp.concatenate(
        [et * N + src, jnp.zeros((EPAD - E,), jnp.int32)])
    dst_p = jnp.concatenate(
        [dst, jnp.full((EPAD - E,), N, jnp.int32)])
    zeros_hbm = jnp.zeros((ZR, D), jnp.float32)

    xpad = jnp.concatenate(
        [x, node_depth.reshape(-1, 1).astype(jnp.int32),
         jnp.zeros((N, 125), jnp.int32)], axis=1)
    ae_pad = jnp.concatenate(
        [params['attr_emb'],
         jnp.zeros((1024 - NUM_NODE_ATTRS, D), jnp.float32)], axis=0)
    h0 = _enc_call(xpad, params['type_emb'], ae_pad, params['depth_emb'])

    states = [h0]
    h = h0
    for l, T in enumerate(LAYER_TIMESTEPS):
        res_list = [states[i] for i in RESIDUALS_MAP.get(l, [])]
        nres = len(res_list)
        wih = params['gru_wih_%d' % l]
        wiaT = wih[:, :D].T
        whhT = params['gru_whh_%d' % l].T
        bih = params['gru_bih_%d' % l].reshape(1, 3 * D)
        bhh = params['gru_bhh_%d' % l].reshape(1, 3 * D)
        res_args = []
        for j, rs in enumerate(res_list):
            res_args.append(rs)
            res_args.append(wih[:, D * (j + 1):D * (j + 2)].T)
        W = params['edge_w_%d' % l]
        for _ in range(T):
            hr = _hr_call(h, W)
            aggp = _edge_pass(hr.reshape(NR * N, D), gidx_p, dst_p,
                              zeros_hbm)
            h = _gru_calls[nres](aggp, h, wiaT, whhT, bih, bhh, *res_args)
        states.append(h)

    batch3 = batch.astype(jnp.int32).reshape(NBLKE, 1, RBE)
    g = _cls_call(h, h0,
                  params['cl_w'][:, :D].T, params['cl_w'][:, D:].T,
                  params['cr_w'][:, :D].T, params['cr_w'][:, D:].T,
                  params['cl_b'].reshape(1, D), params['cr_b'].reshape(1, D),
                  batch3)
    pwT = params['pred_w'].transpose(0, 2, 1)
    pb3 = params['pred_b'].reshape(MAX_SEQ_LEN, 1, NUM_VOCAB)
    return _pred_call(g, pwT, pb3)


# speedup vs baseline: 1.0969x; 1.0969x over previous
import functools

import jax
import jax.numpy as jnp
from jax import lax
from jax.experimental import pallas as pl
from jax.experimental.pallas import tpu as pltpu
from jax.experimental.pallas import tpu_sc as plsc

N = 10000
E = 320000
D = 128
NR = 4
NUM_VOCAB = 5000
MAX_SEQ_LEN = 5
NUM_GRAPHS = 128
NUM_NODE_TYPES = 100
NUM_NODE_ATTRS = 1000
MAX_DEPTH = 20
LAYER_TIMESTEPS = [2, 2, 1, 2, 1]
RESIDUALS_MAP = {2: [0], 4: [0, 2]}

NTILES = 32
K = 112
C0 = 108
C1 = 72
T0 = K * C0
T1 = K * C1
EPAD = 16 * (T0 + T1)
NACC = 10112
ZR = NACC // 16

RB = 1000
NBLK = N // RB
RBE = 200
NBLKE = N // RBE


_sc_mesh = plsc.VectorSubcoreMesh(
    core_axis_name="c", subcore_axis_name="s", num_cores=2, num_subcores=16)


@functools.partial(
    pl.kernel,
    mesh=_sc_mesh,
    out_type=jax.ShapeDtypeStruct((2, NACC, D), jnp.float32),
    scratch_types=[
        pltpu.VMEM((K,), jnp.int32),
        pltpu.VMEM((K,), jnp.int32),
        pltpu.VMEM((K,), jnp.int32),
        pltpu.VMEM((K,), jnp.int32),
        pltpu.VMEM((K, D), jnp.float32),
        pltpu.VMEM((K, D), jnp.float32),
        pltpu.VMEM_SHARED((NACC, D), jnp.float32),
        pltpu.SemaphoreType.DMA,
        pltpu.SemaphoreType.DMA,
        pltpu.SemaphoreType.DMA,
        pltpu.SemaphoreType.DMA,
    ],
)
def _edge_pass(hr_hbm, gidx_hbm, dst_hbm, zeros_hbm, agg_hbm,
               g0_v, d0_v, g1_v, d1_v, rows0_v, rows1_v,
               acc_sh, semg0, semg1, semi0, semi1):
    cid = lax.axis_index("c")
    sid = lax.axis_index("s")
    nchunks = lax.select(cid == 0, C0, C1)
    base = cid * (16 * T0) + sid * lax.select(cid == 0, T0, T1)
    pltpu.sync_copy(zeros_hbm, acc_sh.at[pl.ds(sid * ZR, ZR)])
    plsc.subcore_barrier()

    def idxcpy(c, gbuf, dbuf, sem):
        off = base + c * K
        pltpu.async_copy(gidx_hbm.at[pl.ds(off, K)], gbuf, sem)
        pltpu.async_copy(dst_hbm.at[pl.ds(off, K)], dbuf, sem)

    def idxwait(c, gbuf, dbuf, sem):
        off = base + lax.select(c < nchunks, c, 0) * K
        pltpu.make_async_copy(gidx_hbm.at[pl.ds(off, K)], gbuf, sem).wait()
        pltpu.make_async_copy(dst_hbm.at[pl.ds(off, K)], dbuf, sem).wait()

    idxcpy(0, g0_v, d0_v, semi0)
    idxwait(0, g0_v, d0_v, semi0)
    pltpu.async_copy(hr_hbm.at[g0_v], rows0_v, semg0)
    idxcpy(1, g1_v, d1_v, semi1)

    def halfstep(c, gY, dY, rowsY, semgY, semiY,
                 gX, dX, rowsX, semgX, semiX):
        idxwait(c + 1, gY, dY, semiY)
        pltpu.async_copy(hr_hbm.at[gY], rowsY, semgY)
        pltpu.make_async_copy(hr_hbm.at[gX], rowsX, semgX).wait()
        pltpu.sync_copy(rowsX, acc_sh.at[dX], add=True)
        nxt = lax.select(c + 2 < nchunks, c + 2, 0)
        idxcpy(nxt, gX, dX, semiX)

    def body(j, carry):
        c0 = 2 * j
        halfstep(c0, g1_v, d1_v, rows1_v, semg1, semi1,
                 g0_v, d0_v, rows0_v, semg0, semi0)
        halfstep(c0 + 1, g0_v, d0_v, rows0_v, semg0, semi0,
                 g1_v, d1_v, rows1_v, semg1, semi1)
        return carry

    lax.fori_loop(0, nchunks // 2, body, 0)
    pltpu.make_async_copy(hr_hbm.at[g0_v], rows0_v, semg0).wait()
    idxwait(0, g1_v, d1_v, semi1)
    plsc.subcore_barrier()
    pltpu.sync_copy(acc_sh.at[pl.ds(sid * ZR, ZR)],
                    agg_hbm.at[cid, pl.ds(sid * ZR, ZR)])


def _hr_body(h_ref, w_ref, out_ref):
    out_ref[0] = jnp.dot(h_ref[...], w_ref[0],
                         preferred_element_type=jnp.float32)


_hr_call = pl.pallas_call(
    _hr_body,
    grid=(NR, NBLK),
    in_specs=[
        pl.BlockSpec((RB, D), lambda r, i: (i, 0)),
        pl.BlockSpec((1, D, D), lambda r, i: (r, 0, 0)),
    ],
    out_specs=pl.BlockSpec((1, RB, D), lambda r, i: (r, i, 0)),
    out_shape=jax.ShapeDtypeStruct((NR, N, D), jnp.float32),
)


def _make_gru(nres):
    def body(*refs):
        a_ref, h_ref, wiaT_ref, whhT_ref, bih_ref, bhh_ref = refs[:6]
        res_refs = refs[6:6 + 2 * nres]
        out_ref = refs[6 + 2 * nres]
        agg = a_ref[0] + a_ref[1]
        gi = jnp.dot(agg, wiaT_ref[...],
                     preferred_element_type=jnp.float32) + bih_ref[...]
        for j in range(nres):
            gi = gi + jnp.dot(res_refs[2 * j][...], res_refs[2 * j + 1][...],
                              preferred_element_type=jnp.float32)
        h = h_ref[...]
        gh = jnp.dot(h, whhT_ref[...],
                     preferred_element_type=jnp.float32) + bhh_ref[...]
        r = jax.nn.sigmoid(gi[:, :D] + gh[:, :D])
        z = jax.nn.sigmoid(gi[:, D:2 * D] + gh[:, D:2 * D])
        n = jnp.tanh(gi[:, 2 * D:] + r * gh[:, 2 * D:])
        out_ref[...] = (1.0 - z) * n + z * h

    in_specs = [
        pl.BlockSpec((2, RB, D), lambda i: (0, i, 0)),
        pl.BlockSpec((RB, D), lambda i: (i, 0)),
        pl.BlockSpec((D, 3 * D), lambda i: (0, 0)),
        pl.BlockSpec((D, 3 * D), lambda i: (0, 0)),
        pl.BlockSpec((1, 3 * D), lambda i: (0, 0)),
        pl.BlockSpec((1, 3 * D), lambda i: (0, 0)),
    ]
    for _ in range(nres):
        in_specs.append(pl.BlockSpec((RB, D), lambda i: (i, 0)))
        in_specs.append(pl.BlockSpec((D, 3 * D), lambda i: (0, 0)))
    return pl.pallas_call(
        body,
        grid=(NBLK,),
        in_specs=in_specs,
        out_specs=pl.BlockSpec((RB, D), lambda i: (i, 0)),
        out_shape=jax.ShapeDtypeStruct((N, D), jnp.float32),
    )


_gru_calls = {nres: _make_gru(nres) for nres in (0, 1, 2)}


def _enc_body(x_ref, te_ref, ae_ref, de_ref, out_ref):
    xin = x_ref[...]
    t = xin[:, 0:1]
    oh = (t == lax.broadcasted_iota(jnp.int32, (RBE, NUM_NODE_TYPES), 1))
    h = jnp.dot(oh.astype(jnp.float32), te_ref[...],
                preferred_element_type=jnp.float32,
                precision=lax.Precision.HIGHEST)
    d = xin[:, 2:3]
    ohd = (d == lax.broadcasted_iota(jnp.int32, (RBE, MAX_DEPTH), 1))
    h = h + jnp.dot(ohd.astype(jnp.float32), de_ref[...],
                    preferred_element_type=jnp.float32,
                    precision=lax.Precision.HIGHEST)
    a = xin[:, 1:2]
    for c in range(8):
        ids = lax.broadcasted_iota(jnp.int32, (RBE, 128), 1) + c * 128
        ohc = (a == ids).astype(jnp.float32)
        h = h + jnp.dot(ohc, ae_ref[c * 128:(c + 1) * 128, :],
                        preferred_element_type=jnp.float32,
                        precision=lax.Precision.HIGHEST)
    out_ref[...] = h


_enc_call = pl.pallas_call(
    _enc_body,
    grid=(NBLKE,),
    in_specs=[
        pl.BlockSpec((RBE, 128), lambda i: (i, 0)),
        pl.BlockSpec((NUM_NODE_TYPES, D), lambda i: (0, 0)),
        pl.BlockSpec((1024, D), lambda i: (0, 0)),
        pl.BlockSpec((MAX_DEPTH, D), lambda i: (0, 0)),
    ],
    out_specs=pl.BlockSpec((RBE, D), lambda i: (i, 0)),
    out_shape=jax.ShapeDtypeStruct((N, D), jnp.float32),
)


def _cls_body(h_ref, h0_ref, clw1_ref, clw2_ref, crw1_ref, crw2_ref,
              clb_ref, crb_ref, b_ref, out_ref):
    h = h_ref[...]
    h0 = h0_ref[...]
    t1 = (jnp.dot(h, clw1_ref[...], preferred_element_type=jnp.float32)
          + jnp.dot(h0, clw2_ref[...], preferred_element_type=jnp.float32)
          + clb_ref[...])
    t2 = (jnp.dot(h, crw1_ref[...], preferred_element_type=jnp.float32)
          + jnp.dot(h0, crw2_ref[...], preferred_element_type=jnp.float32)
          + crb_ref[...])
    node_out = jax.nn.sigmoid(t1) * jnp.tanh(t2)
    b = b_ref[0]
    oh = (b == lax.broadcasted_iota(jnp.int32, (NUM_GRAPHS, RBE), 0))

    @pl.when(pl.program_id(0) == 0)
    def _():
        out_ref[...] = jnp.zeros_like(out_ref)

    out_ref[...] += jnp.dot(oh.astype(jnp.float32), node_out,
                            preferred_element_type=jnp.float32,
                            precision=lax.Precision.HIGHEST)


_cls_call = pl.pallas_call(
    _cls_body,
    grid=(NBLKE,),
    in_specs=[
        pl.BlockSpec((RBE, D), lambda i: (i, 0)),
        pl.BlockSpec((RBE, D), lambda i: (i, 0)),
        pl.BlockSpec((D, D), lambda i: (0, 0)),
        pl.BlockSpec((D, D), lambda i: (0, 0)),
        pl.BlockSpec((D, D), lambda i: (0, 0)),
        pl.BlockSpec((D, D), lambda i: (0, 0)),
        pl.BlockSpec((1, D), lambda i: (0, 0)),
        pl.BlockSpec((1, D), lambda i: (0, 0)),
        pl.BlockSpec((1, 1, RBE), lambda i: (i, 0, 0)),
    ],
    out_specs=pl.BlockSpec((NUM_GRAPHS, D), lambda i: (0, 0)),
    out_shape=jax.ShapeDtypeStruct((NUM_GRAPHS, D), jnp.float32),
)

def _pred_body(g_ref, pw_ref, pb_ref, out_ref):
    out_ref[0] = (jnp.dot(g_ref[...], pw_ref[0],
                          preferred_element_type=jnp.float32) + pb_ref[0])


_pred_call = pl.pallas_call(
    _pred_body,
    grid=(MAX_SEQ_LEN,),
    in_specs=[
        pl.BlockSpec((NUM_GRAPHS, D), lambda s: (0, 0)),
        pl.BlockSpec((1, D, NUM_VOCAB), lambda s: (s, 0, 0)),
        pl.BlockSpec((1, 1, NUM_VOCAB), lambda s: (s, 0, 0)),
    ],
    out_specs=pl.BlockSpec((1, NUM_GRAPHS, NUM_VOCAB), lambda s: (s, 0, 0)),
    out_shape=jax.ShapeDtypeStruct((MAX_SEQ_LEN, NUM_GRAPHS, NUM_VOCAB),
                                   jnp.float32),
)


def kernel(x, edge_index, node_depth, batch, edge_attr, params):
    x = x.astype(jnp.int32)
    src = edge_index[0].astype(jnp.int32)
    dst = edge_index[1].astype(jnp.int32)
    et = edge_attr.astype(jnp.int32)

    gidx_p = jnp.concatenate(
        [et * N + src, jnp.zeros((EPAD - E,), jnp.int32)])
    dst_p = jnp.concatenate(
        [dst, jnp.full((EPAD - E,), N, jnp.int32)])
    zeros_hbm = jnp.zeros((ZR, D), jnp.float32)

    xpad = jnp.concatenate(
        [x, node_depth.reshape(-1, 1).astype(jnp.int32),
         jnp.zeros((N, 125), jnp.int32)], axis=1)
    ae_pad = jnp.concatenate(
        [params['attr_emb'],
         jnp.zeros((1024 - NUM_NODE_ATTRS, D), jnp.float32)], axis=0)
    h0 = _enc_call(xpad, params['type_emb'], ae_pad, params['depth_emb'])

    states = [h0]
    h = h0
    for l, T in enumerate(LAYER_TIMESTEPS):
        res_list = [states[i] for i in RESIDUALS_MAP.get(l, [])]
        nres = len(res_list)
        wih = params['gru_wih_%d' % l]
        wiaT = wih[:, :D].T
        whhT = params['gru_whh_%d' % l].T
        bih = params['gru_bih_%d' % l].reshape(1, 3 * D)
        bhh = params['gru_bhh_%d' % l].reshape(1, 3 * D)
        res_args = []
        for j, rs in enumerate(res_list):
            res_args.append(rs)
            res_args.append(wih[:, D * (j + 1):D * (j + 2)].T)
        W = params['edge_w_%d' % l]
        for _ in range(T):
            hr = _hr_call(h, W)
            aggp = _edge_pass(hr.reshape(NR * N, D), gidx_p, dst_p,
                              zeros_hbm)
            h = _gru_calls[nres](aggp, h, wiaT, whhT, bih, bhh, *res_args)
        states.append(h)

    batch3 = batch.astype(jnp.int32).reshape(NBLKE, 1, RBE)
    g = _cls_call(h, h0,
                  params['cl_w'][:, :D].T, params['cl_w'][:, D:].T,
                  params['cr_w'][:, :D].T, params['cr_w'][:, D:].T,
                  params['cl_b'].reshape(1, D), params['cr_b'].reshape(1, D),
                  batch3)
    pwT = params['pred_w'].transpose(0, 2, 1)
    pb3 = params['pred_b'].reshape(MAX_SEQ_LEN, 1, NUM_VOCAB)
    return _pred_call(g, pwT, pb3)


# speedup vs baseline: 1.1294x; 1.0297x over previous
import functools

import jax
import jax.numpy as jnp
from jax import lax
from jax.experimental import pallas as pl
from jax.experimental.pallas import tpu as pltpu
from jax.experimental.pallas import tpu_sc as plsc

N = 10000
E = 320000
D = 128
NR = 4
NUM_VOCAB = 5000
MAX_SEQ_LEN = 5
NUM_GRAPHS = 128
NUM_NODE_TYPES = 100
NUM_NODE_ATTRS = 1000
MAX_DEPTH = 20
LAYER_TIMESTEPS = [2, 2, 1, 2, 1]
RESIDUALS_MAP = {2: [0], 4: [0, 2]}

NTILES = 32
K = 112
C0 = 116
C1 = 64
T0 = K * C0
T1 = K * C1
EPAD = 16 * (T0 + T1)
NACC = 10112
ZR = NACC // 16

RB = 1000
NBLK = N // RB
RBE = 200
NBLKE = N // RBE


_sc_mesh = plsc.VectorSubcoreMesh(
    core_axis_name="c", subcore_axis_name="s", num_cores=2, num_subcores=16)


@functools.partial(
    pl.kernel,
    mesh=_sc_mesh,
    out_type=jax.ShapeDtypeStruct((2, NACC, D), jnp.float32),
    scratch_types=[
        pltpu.VMEM((K,), jnp.int32),
        pltpu.VMEM((K,), jnp.int32),
        pltpu.VMEM((K,), jnp.int32),
        pltpu.VMEM((K,), jnp.int32),
        pltpu.VMEM((K, D), jnp.float32),
        pltpu.VMEM((K, D), jnp.float32),
        pltpu.VMEM_SHARED((NACC, D), jnp.float32),
        pltpu.SemaphoreType.DMA,
        pltpu.SemaphoreType.DMA,
        pltpu.SemaphoreType.DMA,
        pltpu.SemaphoreType.DMA,
    ],
)
def _edge_pass(hr_hbm, gidx_hbm, dst_hbm, zeros_hbm, agg_hbm,
               g0_v, d0_v, g1_v, d1_v, rows0_v, rows1_v,
               acc_sh, semg0, semg1, semi0, semi1):
    cid = lax.axis_index("c")
    sid = lax.axis_index("s")
    nchunks = lax.select(cid == 0, C0, C1)
    base = cid * (16 * T0) + sid * lax.select(cid == 0, T0, T1)
    pltpu.sync_copy(zeros_hbm, acc_sh.at[pl.ds(sid * ZR, ZR)])
    plsc.subcore_barrier()

    def idxcpy(c, gbuf, dbuf, sem):
        off = base + c * K
        pltpu.async_copy(gidx_hbm.at[pl.ds(off, K)], gbuf, sem)
        pltpu.async_copy(dst_hbm.at[pl.ds(off, K)], dbuf, sem)

    def idxwait(c, gbuf, dbuf, sem):
        off = base + lax.select(c < nchunks, c, 0) * K
        pltpu.make_async_copy(gidx_hbm.at[pl.ds(off, K)], gbuf, sem).wait()
        pltpu.make_async_copy(dst_hbm.at[pl.ds(off, K)], dbuf, sem).wait()

    idxcpy(0, g0_v, d0_v, semi0)
    idxwait(0, g0_v, d0_v, semi0)
    pltpu.async_copy(hr_hbm.at[g0_v], rows0_v, semg0)
    idxcpy(1, g1_v, d1_v, semi1)

    def halfstep(c, gY, dY, rowsY, semgY, semiY,
                 gX, dX, rowsX, semgX, semiX):
        idxwait(c + 1, gY, dY, semiY)
        pltpu.async_copy(hr_hbm.at[gY], rowsY, semgY)
        pltpu.make_async_copy(hr_hbm.at[gX], rowsX, semgX).wait()
        pltpu.sync_copy(rowsX, acc_sh.at[dX], add=True)
        nxt = lax.select(c + 2 < nchunks, c + 2, 0)
        idxcpy(nxt, gX, dX, semiX)

    def body(j, carry):
        c0 = 2 * j
        halfstep(c0, g1_v, d1_v, rows1_v, semg1, semi1,
                 g0_v, d0_v, rows0_v, semg0, semi0)
        halfstep(c0 + 1, g0_v, d0_v, rows0_v, semg0, semi0,
                 g1_v, d1_v, rows1_v, semg1, semi1)
        return carry

    lax.fori_loop(0, nchunks // 2, body, 0)
    pltpu.make_async_copy(hr_hbm.at[g0_v], rows0_v, semg0).wait()
    idxwait(0, g1_v, d1_v, semi1)
    plsc.subcore_barrier()
    pltpu.sync_copy(acc_sh.at[pl.ds(sid * ZR, ZR)],
                    agg_hbm.at[cid, pl.ds(sid * ZR, ZR)])


def _hr_body(h_ref, w_ref, out_ref):
    out_ref[0] = jnp.dot(h_ref[...], w_ref[0],
                         preferred_element_type=jnp.float32)


_hr_call = pl.pallas_call(
    _hr_body,
    grid=(NR, NBLK),
    in_specs=[
        pl.BlockSpec((RB, D), lambda r, i: (i, 0)),
        pl.BlockSpec((1, D, D), lambda r, i: (r, 0, 0)),
    ],
    out_specs=pl.BlockSpec((1, RB, D), lambda r, i: (r, i, 0)),
    out_shape=jax.ShapeDtypeStruct((NR, N, D), jnp.float32),
)


def _make_gru(nres):
    def body(*refs):
        a_ref, h_ref, wiaT_ref, whhT_ref, bih_ref, bhh_ref = refs[:6]
        res_refs = refs[6:6 + 2 * nres]
        out_ref = refs[6 + 2 * nres]
        agg = a_ref[0] + a_ref[1]
        gi = jnp.dot(agg, wiaT_ref[...],
                     preferred_element_type=jnp.float32) + bih_ref[...]
        for j in range(nres):
            gi = gi + jnp.dot(res_refs[2 * j][...], res_refs[2 * j + 1][...],
                              preferred_element_type=jnp.float32)
        h = h_ref[...]
        gh = jnp.dot(h, whhT_ref[...],
                     preferred_element_type=jnp.float32) + bhh_ref[...]
        r = jax.nn.sigmoid(gi[:, :D] + gh[:, :D])
        z = jax.nn.sigmoid(gi[:, D:2 * D] + gh[:, D:2 * D])
        n = jnp.tanh(gi[:, 2 * D:] + r * gh[:, 2 * D:])
        out_ref[...] = (1.0 - z) * n + z * h

    in_specs = [
        pl.BlockSpec((2, RB, D), lambda i: (0, i, 0)),
        pl.BlockSpec((RB, D), lambda i: (i, 0)),
        pl.BlockSpec((D, 3 * D), lambda i: (0, 0)),
        pl.BlockSpec((D, 3 * D), lambda i: (0, 0)),
        pl.BlockSpec((1, 3 * D), lambda i: (0, 0)),
        pl.BlockSpec((1, 3 * D), lambda i: (0, 0)),
    ]
    for _ in range(nres):
        in_specs.append(pl.BlockSpec((RB, D), lambda i: (i, 0)))
        in_specs.append(pl.BlockSpec((D, 3 * D), lambda i: (0, 0)))
    return pl.pallas_call(
        body,
        grid=(NBLK,),
        in_specs=in_specs,
        out_specs=pl.BlockSpec((RB, D), lambda i: (i, 0)),
        out_shape=jax.ShapeDtypeStruct((N, D), jnp.float32),
    )


_gru_calls = {nres: _make_gru(nres) for nres in (0, 1, 2)}


def _enc_body(x_ref, te_ref, ae_ref, de_ref, out_ref):
    xin = x_ref[...]
    t = xin[:, 0:1]
    oh = (t == lax.broadcasted_iota(jnp.int32, (RBE, NUM_NODE_TYPES), 1))
    h = jnp.dot(oh.astype(jnp.float32), te_ref[...],
                preferred_element_type=jnp.float32,
                precision=lax.Precision.HIGHEST)
    d = xin[:, 2:3]
    ohd = (d == lax.broadcasted_iota(jnp.int32, (RBE, MAX_DEPTH), 1))
    h = h + jnp.dot(ohd.astype(jnp.float32), de_ref[...],
                    preferred_element_type=jnp.float32,
                    precision=lax.Precision.HIGHEST)
    a = xin[:, 1:2]
    for c in range(8):
        ids = lax.broadcasted_iota(jnp.int32, (RBE, 128), 1) + c * 128
        ohc = (a == ids).astype(jnp.float32)
        h = h + jnp.dot(ohc, ae_ref[c * 128:(c + 1) * 128, :],
                        preferred_element_type=jnp.float32,
                        precision=lax.Precision.HIGHEST)
    out_ref[...] = h


_enc_call = pl.pallas_call(
    _enc_body,
    grid=(NBLKE,),
    in_specs=[
        pl.BlockSpec((RBE, 128), lambda i: (i, 0)),
        pl.BlockSpec((NUM_NODE_TYPES, D), lambda i: (0, 0)),
        pl.BlockSpec((1024, D), lambda i: (0, 0)),
        pl.BlockSpec((MAX_DEPTH, D), lambda i: (0, 0)),
    ],
    out_specs=pl.BlockSpec((RBE, D), lambda i: (i, 0)),
    out_shape=jax.ShapeDtypeStruct((N, D), jnp.float32),
)


def _cls_body(h_ref, h0_ref, clw1_ref, clw2_ref, crw1_ref, crw2_ref,
              clb_ref, crb_ref, b_ref, out_ref):
    h = h_ref[...]
    h0 = h0_ref[...]
    t1 = (jnp.dot(h, clw1_ref[...], preferred_element_type=jnp.float32)
          + jnp.dot(h0, clw2_ref[...], preferred_element_type=jnp.float32)
          + clb_ref[...])
    t2 = (jnp.dot(h, crw1_ref[...], preferred_element_type=jnp.float32)
          + jnp.dot(h0, crw2_ref[...], preferred_element_type=jnp.float32)
          + crb_ref[...])
    node_out = jax.nn.sigmoid(t1) * jnp.tanh(t2)
    b = b_ref[0]
    oh = (b == lax.broadcasted_iota(jnp.int32, (NUM_GRAPHS, RBE), 0))

    @pl.when(pl.program_id(0) == 0)
    def _():
        out_ref[...] = jnp.zeros_like(out_ref)

    out_ref[...] += jnp.dot(oh.astype(jnp.float32), node_out,
                            preferred_element_type=jnp.float32,
                            precision=lax.Precision.HIGHEST)


_cls_call = pl.pallas_call(
    _cls_body,
    grid=(NBLKE,),
    in_specs=[
        pl.BlockSpec((RBE, D), lambda i: (i, 0)),
        pl.BlockSpec((RBE, D), lambda i: (i, 0)),
        pl.BlockSpec((D, D), lambda i: (0, 0)),
        pl.BlockSpec((D, D), lambda i: (0, 0)),
        pl.BlockSpec((D, D), lambda i: (0, 0)),
        pl.BlockSpec((D, D), lambda i: (0, 0)),
        pl.BlockSpec((1, D), lambda i: (0, 0)),
        pl.BlockSpec((1, D), lambda i: (0, 0)),
        pl.BlockSpec((1, 1, RBE), lambda i: (i, 0, 0)),
    ],
    out_specs=pl.BlockSpec((NUM_GRAPHS, D), lambda i: (0, 0)),
    out_shape=jax.ShapeDtypeStruct((NUM_GRAPHS, D), jnp.float32),
)

def _pred_body(g_ref, pw_ref, pb_ref, out_ref):
    out_ref[0] = (jnp.dot(g_ref[...], pw_ref[0],
                          preferred_element_type=jnp.float32) + pb_ref[0])


_pred_call = pl.pallas_call(
    _pred_body,
    grid=(MAX_SEQ_LEN,),
    in_specs=[
        pl.BlockSpec((NUM_GRAPHS, D), lambda s: (0, 0)),
        pl.BlockSpec((1, D, NUM_VOCAB), lambda s: (s, 0, 0)),
        pl.BlockSpec((1, 1, NUM_VOCAB), lambda s: (s, 0, 0)),
    ],
    out_specs=pl.BlockSpec((1, NUM_GRAPHS, NUM_VOCAB), lambda s: (s, 0, 0)),
    out_shape=jax.ShapeDtypeStruct((MAX_SEQ_LEN, NUM_GRAPHS, NUM_VOCAB),
                                   jnp.float32),
)


def kernel(x, edge_index, node_depth, batch, edge_attr, params):
    x = x.astype(jnp.int32)
    src = edge_index[0].astype(jnp.int32)
    dst = edge_index[1].astype(jnp.int32)
    et = edge_attr.astype(jnp.int32)

    gidx_p = jnp.concatenate(
        [et * N + src, jnp.zeros((EPAD - E,), jnp.int32)])
    dst_p = jnp.concatenate(
        [dst, jnp.full((EPAD - E,), N, jnp.int32)])
    zeros_hbm = jnp.zeros((ZR, D), jnp.float32)

    xpad = jnp.concatenate(
        [x, node_depth.reshape(-1, 1).astype(jnp.int32),
         jnp.zeros((N, 125), jnp.int32)], axis=1)
    ae_pad = jnp.concatenate(
        [params['attr_emb'],
         jnp.zeros((1024 - NUM_NODE_ATTRS, D), jnp.float32)], axis=0)
    h0 = _enc_call(xpad, params['type_emb'], ae_pad, params['depth_emb'])

    states = [h0]
    h = h0
    for l, T in enumerate(LAYER_TIMESTEPS):
        res_list = [states[i] for i in RESIDUALS_MAP.get(l, [])]
        nres = len(res_list)
        wih = params['gru_wih_%d' % l]
        wiaT = wih[:, :D].T
        whhT = params['gru_whh_%d' % l].T
        bih = params['gru_bih_%d' % l].reshape(1, 3 * D)
        bhh = params['gru_bhh_%d' % l].reshape(1, 3 * D)
        res_args = []
        for j, rs in enumerate(res_list):
            res_args.append(rs)
            res_args.append(wih[:, D * (j + 1):D * (j + 2)].T)
        W = params['edge_w_%d' % l]
        for _ in range(T):
            hr = _hr_call(h, W)
            aggp = _edge_pass(hr.reshape(NR * N, D), gidx_p, dst_p,
                              zeros_hbm)
            h = _gru_calls[nres](aggp, h, wiaT, whhT, bih, bhh, *res_args)
        states.append(h)

    batch3 = batch.astype(jnp.int32).reshape(NBLKE, 1, RBE)
    g = _cls_call(h, h0,
                  params['cl_w'][:, :D].T, params['cl_w'][:, D:].T,
                  params['cr_w'][:, :D].T, params['cr_w'][:, D:].T,
                  params['cl_b'].reshape(1, D), params['cr_b'].reshape(1, D),
                  batch3)
    pwT = params['pred_w'].transpose(0, 2, 1)
    pb3 = params['pred_b'].reshape(MAX_SEQ_LEN, 1, NUM_VOCAB)
    return _pred_call(g, pwT, pb3)


# speedup vs baseline: 1.1602x; 1.0273x over previous
import functools

import jax
import jax.numpy as jnp
from jax import lax
from jax.experimental import pallas as pl
from jax.experimental.pallas import tpu as pltpu
from jax.experimental.pallas import tpu_sc as plsc

N = 10000
E = 320000
D = 128
NR = 4
NUM_VOCAB = 5000
MAX_SEQ_LEN = 5
NUM_GRAPHS = 128
NUM_NODE_TYPES = 100
NUM_NODE_ATTRS = 1000
MAX_DEPTH = 20
LAYER_TIMESTEPS = [2, 2, 1, 2, 1]
RESIDUALS_MAP = {2: [0], 4: [0, 2]}

NTILES = 32
K = 112
C0 = 124
C1 = 56
T0 = K * C0
T1 = K * C1
EPAD = 16 * (T0 + T1)
NACC = 10112
ZR = NACC // 16

RB = 1000
NBLK = N // RB
RBE = 200
NBLKE = N // RBE


_sc_mesh = plsc.VectorSubcoreMesh(
    core_axis_name="c", subcore_axis_name="s", num_cores=2, num_subcores=16)


@functools.partial(
    pl.kernel,
    mesh=_sc_mesh,
    out_type=jax.ShapeDtypeStruct((2, NACC, D), jnp.float32),
    scratch_types=[
        pltpu.VMEM((K,), jnp.int32),
        pltpu.VMEM((K,), jnp.int32),
        pltpu.VMEM((K,), jnp.int32),
        pltpu.VMEM((K,), jnp.int32),
        pltpu.VMEM((K, D), jnp.float32),
        pltpu.VMEM((K, D), jnp.float32),
        pltpu.VMEM_SHARED((NACC, D), jnp.float32),
        pltpu.SemaphoreType.DMA,
        pltpu.SemaphoreType.DMA,
        pltpu.SemaphoreType.DMA,
        pltpu.SemaphoreType.DMA,
    ],
)
def _edge_pass(hr_hbm, gidx_hbm, dst_hbm, zeros_hbm, agg_hbm,
               g0_v, d0_v, g1_v, d1_v, rows0_v, rows1_v,
               acc_sh, semg0, semg1, semi0, semi1):
    cid = lax.axis_index("c")
    sid = lax.axis_index("s")
    nchunks = lax.select(cid == 0, C0, C1)
    base = cid * (16 * T0) + sid * lax.select(cid == 0, T0, T1)
    pltpu.sync_copy(zeros_hbm, acc_sh.at[pl.ds(sid * ZR, ZR)])
    plsc.subcore_barrier()

    def idxcpy(c, gbuf, dbuf, sem):
        off = base + c * K
        pltpu.async_copy(gidx_hbm.at[pl.ds(off, K)], gbuf, sem)
        pltpu.async_copy(dst_hbm.at[pl.ds(off, K)], dbuf, sem)

    def idxwait(c, gbuf, dbuf, sem):
        off = base + lax.select(c < nchunks, c, 0) * K
        pltpu.make_async_copy(gidx_hbm.at[pl.ds(off, K)], gbuf, sem).wait()
        pltpu.make_async_copy(dst_hbm.at[pl.ds(off, K)], dbuf, sem).wait()

    idxcpy(0, g0_v, d0_v, semi0)
    idxwait(0, g0_v, d0_v, semi0)
    pltpu.async_copy(hr_hbm.at[g0_v], rows0_v, semg0)
    idxcpy(1, g1_v, d1_v, semi1)

    def halfstep(c, gY, dY, rowsY, semgY, semiY,
                 gX, dX, rowsX, semgX, semiX):
        idxwait(c + 1, gY, dY, semiY)
        pltpu.async_copy(hr_hbm.at[gY], rowsY, semgY)
        pltpu.make_async_copy(hr_hbm.at[gX], rowsX, semgX).wait()
        pltpu.sync_copy(rowsX, acc_sh.at[dX], add=True)
        nxt = lax.select(c + 2 < nchunks, c + 2, 0)
        idxcpy(nxt, gX, dX, semiX)

    def body(j, carry):
        c0 = 2 * j
        halfstep(c0, g1_v, d1_v, rows1_v, semg1, semi1,
                 g0_v, d0_v, rows0_v, semg0, semi0)
        halfstep(c0 + 1, g0_v, d0_v, rows0_v, semg0, semi0,
                 g1_v, d1_v, rows1_v, semg1, semi1)
        return carry

    lax.fori_loop(0, nchunks // 2, body, 0)
    pltpu.make_async_copy(hr_hbm.at[g0_v], rows0_v, semg0).wait()
    idxwait(0, g1_v, d1_v, semi1)
    plsc.subcore_barrier()
    pltpu.sync_copy(acc_sh.at[pl.ds(sid * ZR, ZR)],
                    agg_hbm.at[cid, pl.ds(sid * ZR, ZR)])


def _hr_body(h_ref, w_ref, out_ref):
    out_ref[0] = jnp.dot(h_ref[...], w_ref[0],
                         preferred_element_type=jnp.float32)


_hr_call = pl.pallas_call(
    _hr_body,
    grid=(NR, NBLK),
    in_specs=[
        pl.BlockSpec((RB, D), lambda r, i: (i, 0)),
        pl.BlockSpec((1, D, D), lambda r, i: (r, 0, 0)),
    ],
    out_specs=pl.BlockSpec((1, RB, D), lambda r, i: (r, i, 0)),
    out_shape=jax.ShapeDtypeStruct((NR, N, D), jnp.float32),
)


def _make_gru(nres):
    def body(*refs):
        a_ref, h_ref, wiaT_ref, whhT_ref, bih_ref, bhh_ref = refs[:6]
        res_refs = refs[6:6 + 2 * nres]
        out_ref = refs[6 + 2 * nres]
        agg = a_ref[0] + a_ref[1]
        gi = jnp.dot(agg, wiaT_ref[...],
                     preferred_element_type=jnp.float32) + bih_ref[...]
        for j in range(nres):
            gi = gi + jnp.dot(res_refs[2 * j][...], res_refs[2 * j + 1][...],
                              preferred_element_type=jnp.float32)
        h = h_ref[...]
        gh = jnp.dot(h, whhT_ref[...],
                     preferred_element_type=jnp.float32) + bhh_ref[...]
        r = jax.nn.sigmoid(gi[:, :D] + gh[:, :D])
        z = jax.nn.sigmoid(gi[:, D:2 * D] + gh[:, D:2 * D])
        n = jnp.tanh(gi[:, 2 * D:] + r * gh[:, 2 * D:])
        out_ref[...] = (1.0 - z) * n + z * h

    in_specs = [
        pl.BlockSpec((2, RB, D), lambda i: (0, i, 0)),
        pl.BlockSpec((RB, D), lambda i: (i, 0)),
        pl.BlockSpec((D, 3 * D), lambda i: (0, 0)),
        pl.BlockSpec((D, 3 * D), lambda i: (0, 0)),
        pl.BlockSpec((1, 3 * D), lambda i: (0, 0)),
        pl.BlockSpec((1, 3 * D), lambda i: (0, 0)),
    ]
    for _ in range(nres):
        in_specs.append(pl.BlockSpec((RB, D), lambda i: (i, 0)))
        in_specs.append(pl.BlockSpec((D, 3 * D), lambda i: (0, 0)))
    return pl.pallas_call(
        body,
        grid=(NBLK,),
        in_specs=in_specs,
        out_specs=pl.BlockSpec((RB, D), lambda i: (i, 0)),
        out_shape=jax.ShapeDtypeStruct((N, D), jnp.float32),
    )


_gru_calls = {nres: _make_gru(nres) for nres in (0, 1, 2)}


def _enc_body(x_ref, te_ref, ae_ref, de_ref, out_ref):
    xin = x_ref[...]
    t = xin[:, 0:1]
    oh = (t == lax.broadcasted_iota(jnp.int32, (RBE, NUM_NODE_TYPES), 1))
    h = jnp.dot(oh.astype(jnp.float32), te_ref[...],
                preferred_element_type=jnp.float32,
                precision=lax.Precision.HIGHEST)
    d = xin[:, 2:3]
    ohd = (d == lax.broadcasted_iota(jnp.int32, (RBE, MAX_DEPTH), 1))
    h = h + jnp.dot(ohd.astype(jnp.float32), de_ref[...],
                    preferred_element_type=jnp.float32,
                    precision=lax.Precision.HIGHEST)
    a = xin[:, 1:2]
    for c in range(8):
        ids = lax.broadcasted_iota(jnp.int32, (RBE, 128), 1) + c * 128
        ohc = (a == ids).astype(jnp.float32)
        h = h + jnp.dot(ohc, ae_ref[c * 128:(c + 1) * 128, :],
                        preferred_element_type=jnp.float32,
                        precision=lax.Precision.HIGHEST)
    out_ref[...] = h


_enc_call = pl.pallas_call(
    _enc_body,
    grid=(NBLKE,),
    in_specs=[
        pl.BlockSpec((RBE, 128), lambda i: (i, 0)),
        pl.BlockSpec((NUM_NODE_TYPES, D), lambda i: (0, 0)),
        pl.BlockSpec((1024, D), lambda i: (0, 0)),
        pl.BlockSpec((MAX_DEPTH, D), lambda i: (0, 0)),
    ],
    out_specs=pl.BlockSpec((RBE, D), lambda i: (i, 0)),
    out_shape=jax.ShapeDtypeStruct((N, D), jnp.float32),
)


def _cls_body(h_ref, h0_ref, clw1_ref, clw2_ref, crw1_ref, crw2_ref,
              clb_ref, crb_ref, b_ref, out_ref):
    h = h_ref[...]
    h0 = h0_ref[...]
    t1 = (jnp.dot(h, clw1_ref[...], preferred_element_type=jnp.float32)
          + jnp.dot(h0, clw2_ref[...], preferred_element_type=jnp.float32)
          + clb_ref[...])
    t2 = (jnp.dot(h, crw1_ref[...], preferred_element_type=jnp.float32)
          + jnp.dot(h0, crw2_ref[...], preferred_element_type=jnp.float32)
          + crb_ref[...])
    node_out = jax.nn.sigmoid(t1) * jnp.tanh(t2)
    b = b_ref[0]
    oh = (b == lax.broadcasted_iota(jnp.int32, (NUM_GRAPHS, RBE), 0))

    @pl.when(pl.program_id(0) == 0)
    def _():
        out_ref[...] = jnp.zeros_like(out_ref)

    out_ref[...] += jnp.dot(oh.astype(jnp.float32), node_out,
                            preferred_element_type=jnp.float32,
                            precision=lax.Precision.HIGHEST)


_cls_call = pl.pallas_call(
    _cls_body,
    grid=(NBLKE,),
    in_specs=[
        pl.BlockSpec((RBE, D), lambda i: (i, 0)),
        pl.BlockSpec((RBE, D), lambda i: (i, 0)),
        pl.BlockSpec((D, D), lambda i: (0, 0)),
        pl.BlockSpec((D, D), lambda i: (0, 0)),
        pl.BlockSpec((D, D), lambda i: (0, 0)),
        pl.BlockSpec((D, D), lambda i: (0, 0)),
        pl.BlockSpec((1, D), lambda i: (0, 0)),
        pl.BlockSpec((1, D), lambda i: (0, 0)),
        pl.BlockSpec((1, 1, RBE), lambda i: (i, 0, 0)),
    ],
    out_specs=pl.BlockSpec((NUM_GRAPHS, D), lambda i: (0, 0)),
    out_shape=jax.ShapeDtypeStruct((NUM_GRAPHS, D), jnp.float32),
)

def _pred_body(g_ref, pw_ref, pb_ref, out_ref):
    out_ref[0] = (jnp.dot(g_ref[...], pw_ref[0],
                          preferred_element_type=jnp.float32) + pb_ref[0])


_pred_call = pl.pallas_call(
    _pred_body,
    grid=(MAX_SEQ_LEN,),
    in_specs=[
        pl.BlockSpec((NUM_GRAPHS, D), lambda s: (0, 0)),
        pl.BlockSpec((1, D, NUM_VOCAB), lambda s: (s, 0, 0)),
        pl.BlockSpec((1, 1, NUM_VOCAB), lambda s: (s, 0, 0)),
    ],
    out_specs=pl.BlockSpec((1, NUM_GRAPHS, NUM_VOCAB), lambda s: (s, 0, 0)),
    out_shape=jax.ShapeDtypeStruct((MAX_SEQ_LEN, NUM_GRAPHS, NUM_VOCAB),
                                   jnp.float32),
)


def kernel(x, edge_index, node_depth, batch, edge_attr, params):
    x = x.astype(jnp.int32)
    src = edge_index[0].astype(jnp.int32)
    dst = edge_index[1].astype(jnp.int32)
    et = edge_attr.astype(jnp.int32)

    gidx_p = jnp.concatenate(
        [et * N + src, jnp.zeros((EPAD - E,), jnp.int32)])
    dst_p = jnp.concatenate(
        [dst, jnp.full((EPAD - E,), N, jnp.int32)])
    zeros_hbm = jnp.zeros((ZR, D), jnp.float32)

    xpad = jnp.concatenate(
        [x, node_depth.reshape(-1, 1).astype(jnp.int32),
         jnp.zeros((N, 125), jnp.int32)], axis=1)
    ae_pad = jnp.concatenate(
        [params['attr_emb'],
         jnp.zeros((1024 - NUM_NODE_ATTRS, D), jnp.float32)], axis=0)
    h0 = _enc_call(xpad, params['type_emb'], ae_pad, params['depth_emb'])

    states = [h0]
    h = h0
    for l, T in enumerate(LAYER_TIMESTEPS):
        res_list = [states[i] for i in RESIDUALS_MAP.get(l, [])]
        nres = len(res_list)
        wih = params['gru_wih_%d' % l]
        wiaT = wih[:, :D].T
        whhT = params['gru_whh_%d' % l].T
        bih = params['gru_bih_%d' % l].reshape(1, 3 * D)
        bhh = params['gru_bhh_%d' % l].reshape(1, 3 * D)
        res_args = []
        for j, rs in enumerate(res_list):
            res_args.append(rs)
            res_args.append(wih[:, D * (j + 1):D * (j + 2)].T)
        W = params['edge_w_%d' % l]
        for _ in range(T):
            hr = _hr_call(h, W)
            aggp = _edge_pass(hr.reshape(NR * N, D), gidx_p, dst_p,
                              zeros_hbm)
            h = _gru_calls[nres](aggp, h, wiaT, whhT, bih, bhh, *res_args)
        states.append(h)

    batch3 = batch.astype(jnp.int32).reshape(NBLKE, 1, RBE)
    g = _cls_call(h, h0,
                  params['cl_w'][:, :D].T, params['cl_w'][:, D:].T,
                  params['cr_w'][:, :D].T, params['cr_w'][:, D:].T,
                  params['cl_b'].reshape(1, D), params['cr_b'].reshape(1, D),
                  batch3)
    pwT = params['pred_w'].transpose(0, 2, 1)
    pb3 = params['pred_b'].reshape(MAX_SEQ_LEN, 1, NUM_VOCAB)
    return _pred_call(g, pwT, pb3)


# speedup vs baseline: 1.1668x; 1.0056x over previous
import functools

import jax
import jax.numpy as jnp
from jax import lax
from jax.experimental import pallas as pl
from jax.experimental.pallas import tpu as pltpu
from jax.experimental.pallas import tpu_sc as plsc

N = 10000
E = 320000
D = 128
NR = 4
NUM_VOCAB = 5000
MAX_SEQ_LEN = 5
NUM_GRAPHS = 128
NUM_NODE_TYPES = 100
NUM_NODE_ATTRS = 1000
MAX_DEPTH = 20
LAYER_TIMESTEPS = [2, 2, 1, 2, 1]
RESIDUALS_MAP = {2: [0], 4: [0, 2]}

NTILES = 32
K = 112
C0 = 132
C1 = 48
T0 = K * C0
T1 = K * C1
EPAD = 16 * (T0 + T1)
NACC = 10112
ZR = NACC // 16

RB = 1000
NBLK = N // RB
RBE = 200
NBLKE = N // RBE


_sc_mesh = plsc.VectorSubcoreMesh(
    core_axis_name="c", subcore_axis_name="s", num_cores=2, num_subcores=16)


@functools.partial(
    pl.kernel,
    mesh=_sc_mesh,
    out_type=jax.ShapeDtypeStruct((2, NACC, D), jnp.float32),
    scratch_types=[
        pltpu.VMEM((K,), jnp.int32),
        pltpu.VMEM((K,), jnp.int32),
        pltpu.VMEM((K,), jnp.int32),
        pltpu.VMEM((K,), jnp.int32),
        pltpu.VMEM((K, D), jnp.float32),
        pltpu.VMEM((K, D), jnp.float32),
        pltpu.VMEM_SHARED((NACC, D), jnp.float32),
        pltpu.SemaphoreType.DMA,
        pltpu.SemaphoreType.DMA,
        pltpu.SemaphoreType.DMA,
        pltpu.SemaphoreType.DMA,
    ],
)
def _edge_pass(hr_hbm, gidx_hbm, dst_hbm, zeros_hbm, agg_hbm,
               g0_v, d0_v, g1_v, d1_v, rows0_v, rows1_v,
               acc_sh, semg0, semg1, semi0, semi1):
    cid = lax.axis_index("c")
    sid = lax.axis_index("s")
    nchunks = lax.select(cid == 0, C0, C1)
    base = cid * (16 * T0) + sid * lax.select(cid == 0, T0, T1)
    pltpu.sync_copy(zeros_hbm, acc_sh.at[pl.ds(sid * ZR, ZR)])
    plsc.subcore_barrier()

    def idxcpy(c, gbuf, dbuf, sem):
        off = base + c * K
        pltpu.async_copy(gidx_hbm.at[pl.ds(off, K)], gbuf, sem)
        pltpu.async_copy(dst_hbm.at[pl.ds(off, K)], dbuf, sem)

    def idxwait(c, gbuf, dbuf, sem):
        off = base + lax.select(c < nchunks, c, 0) * K
        pltpu.make_async_copy(gidx_hbm.at[pl.ds(off, K)], gbuf, sem).wait()
        pltpu.make_async_copy(dst_hbm.at[pl.ds(off, K)], dbuf, sem).wait()

    idxcpy(0, g0_v, d0_v, semi0)
    idxwait(0, g0_v, d0_v, semi0)
    pltpu.async_copy(hr_hbm.at[g0_v], rows0_v, semg0)
    idxcpy(1, g1_v, d1_v, semi1)

    def halfstep(c, gY, dY, rowsY, semgY, semiY,
                 gX, dX, rowsX, semgX, semiX):
        idxwait(c + 1, gY, dY, semiY)
        pltpu.async_copy(hr_hbm.at[gY], rowsY, semgY)
        pltpu.make_async_copy(hr_hbm.at[gX], rowsX, semgX).wait()
        pltpu.sync_copy(rowsX, acc_sh.at[dX], add=True)
        nxt = lax.select(c + 2 < nchunks, c + 2, 0)
        idxcpy(nxt, gX, dX, semiX)

    def body(j, carry):
        c0 = 2 * j
        halfstep(c0, g1_v, d1_v, rows1_v, semg1, semi1,
                 g0_v, d0_v, rows0_v, semg0, semi0)
        halfstep(c0 + 1, g0_v, d0_v, rows0_v, semg0, semi0,
                 g1_v, d1_v, rows1_v, semg1, semi1)
        return carry

    lax.fori_loop(0, nchunks // 2, body, 0)
    pltpu.make_async_copy(hr_hbm.at[g0_v], rows0_v, semg0).wait()
    idxwait(0, g1_v, d1_v, semi1)
    plsc.subcore_barrier()
    pltpu.sync_copy(acc_sh.at[pl.ds(sid * ZR, ZR)],
                    agg_hbm.at[cid, pl.ds(sid * ZR, ZR)])


def _hr_body(h_ref, w_ref, out_ref):
    out_ref[0] = jnp.dot(h_ref[...], w_ref[0],
                         preferred_element_type=jnp.float32)


_hr_call = pl.pallas_call(
    _hr_body,
    grid=(NR, NBLK),
    in_specs=[
        pl.BlockSpec((RB, D), lambda r, i: (i, 0)),
        pl.BlockSpec((1, D, D), lambda r, i: (r, 0, 0)),
    ],
    out_specs=pl.BlockSpec((1, RB, D), lambda r, i: (r, i, 0)),
    out_shape=jax.ShapeDtypeStruct((NR, N, D), jnp.float32),
)


def _make_gru(nres):
    def body(*refs):
        a_ref, h_ref, wiaT_ref, whhT_ref, bih_ref, bhh_ref = refs[:6]
        res_refs = refs[6:6 + 2 * nres]
        out_ref = refs[6 + 2 * nres]
        agg = a_ref[0] + a_ref[1]
        gi = jnp.dot(agg, wiaT_ref[...],
                     preferred_element_type=jnp.float32) + bih_ref[...]
        for j in range(nres):
            gi = gi + jnp.dot(res_refs[2 * j][...], res_refs[2 * j + 1][...],
                              preferred_element_type=jnp.float32)
        h = h_ref[...]
        gh = jnp.dot(h, whhT_ref[...],
                     preferred_element_type=jnp.float32) + bhh_ref[...]
        r = jax.nn.sigmoid(gi[:, :D] + gh[:, :D])
        z = jax.nn.sigmoid(gi[:, D:2 * D] + gh[:, D:2 * D])
        n = jnp.tanh(gi[:, 2 * D:] + r * gh[:, 2 * D:])
        out_ref[...] = (1.0 - z) * n + z * h

    in_specs = [
        pl.BlockSpec((2, RB, D), lambda i: (0, i, 0)),
        pl.BlockSpec((RB, D), lambda i: (i, 0)),
        pl.BlockSpec((D, 3 * D), lambda i: (0, 0)),
        pl.BlockSpec((D, 3 * D), lambda i: (0, 0)),
        pl.BlockSpec((1, 3 * D), lambda i: (0, 0)),
        pl.BlockSpec((1, 3 * D), lambda i: (0, 0)),
    ]
    for _ in range(nres):
        in_specs.append(pl.BlockSpec((RB, D), lambda i: (i, 0)))
        in_specs.append(pl.BlockSpec((D, 3 * D), lambda i: (0, 0)))
    return pl.pallas_call(
        body,
        grid=(NBLK,),
        in_specs=in_specs,
        out_specs=pl.BlockSpec((RB, D), lambda i: (i, 0)),
        out_shape=jax.ShapeDtypeStruct((N, D), jnp.float32),
    )


_gru_calls = {nres: _make_gru(nres) for nres in (0, 1, 2)}


def _enc_body(x_ref, te_ref, ae_ref, de_ref, out_ref):
    xin = x_ref[...]
    t = xin[:, 0:1]
    oh = (t == lax.broadcasted_iota(jnp.int32, (RBE, NUM_NODE_TYPES), 1))
    h = jnp.dot(oh.astype(jnp.float32), te_ref[...],
                preferred_element_type=jnp.float32,
                precision=lax.Precision.HIGHEST)
    d = xin[:, 2:3]
    ohd = (d == lax.broadcasted_iota(jnp.int32, (RBE, MAX_DEPTH), 1))
    h = h + jnp.dot(ohd.astype(jnp.float32), de_ref[...],
                    preferred_element_type=jnp.float32,
                    precision=lax.Precision.HIGHEST)
    a = xin[:, 1:2]
    for c in range(8):
        ids = lax.broadcasted_iota(jnp.int32, (RBE, 128), 1) + c * 128
        ohc = (a == ids).astype(jnp.float32)
        h = h + jnp.dot(ohc, ae_ref[c * 128:(c + 1) * 128, :],
                        preferred_element_type=jnp.float32,
                        precision=lax.Precision.HIGHEST)
    out_ref[...] = h


_enc_call = pl.pallas_call(
    _enc_body,
    grid=(NBLKE,),
    in_specs=[
        pl.BlockSpec((RBE, 128), lambda i: (i, 0)),
        pl.BlockSpec((NUM_NODE_TYPES, D), lambda i: (0, 0)),
        pl.BlockSpec((1024, D), lambda i: (0, 0)),
        pl.BlockSpec((MAX_DEPTH, D), lambda i: (0, 0)),
    ],
    out_specs=pl.BlockSpec((RBE, D), lambda i: (i, 0)),
    out_shape=jax.ShapeDtypeStruct((N, D), jnp.float32),
)


def _cls_body(h_ref, h0_ref, clw1_ref, clw2_ref, crw1_ref, crw2_ref,
              clb_ref, crb_ref, b_ref, out_ref):
    h = h_ref[...]
    h0 = h0_ref[...]
    t1 = (jnp.dot(h, clw1_ref[...], preferred_element_type=jnp.float32)
          + jnp.dot(h0, clw2_ref[...], preferred_element_type=jnp.float32)
          + clb_ref[...])
    t2 = (jnp.dot(h, crw1_ref[...], preferred_element_type=jnp.float32)
          + jnp.dot(h0, crw2_ref[...], preferred_element_type=jnp.float32)
          + crb_ref[...])
    node_out = jax.nn.sigmoid(t1) * jnp.tanh(t2)
    b = b_ref[0]
    oh = (b == lax.broadcasted_iota(jnp.int32, (NUM_GRAPHS, RBE), 0))

    @pl.when(pl.program_id(0) == 0)
    def _():
        out_ref[...] = jnp.zeros_like(out_ref)

    out_ref[...] += jnp.dot(oh.astype(jnp.float32), node_out,
                            preferred_element_type=jnp.float32,
                            precision=lax.Precision.HIGHEST)


_cls_call = pl.pallas_call(
    _cls_body,
    grid=(NBLKE,),
    in_specs=[
        pl.BlockSpec((RBE, D), lambda i: (i, 0)),
        pl.BlockSpec((RBE, D), lambda i: (i, 0)),
        pl.BlockSpec((D, D), lambda i: (0, 0)),
        pl.BlockSpec((D, D), lambda i: (0, 0)),
        pl.BlockSpec((D, D), lambda i: (0, 0)),
        pl.BlockSpec((D, D), lambda i: (0, 0)),
        pl.BlockSpec((1, D), lambda i: (0, 0)),
        pl.BlockSpec((1, D), lambda i: (0, 0)),
        pl.BlockSpec((1, 1, RBE), lambda i: (i, 0, 0)),
    ],
    out_specs=pl.BlockSpec((NUM_GRAPHS, D), lambda i: (0, 0)),
    out_shape=jax.ShapeDtypeStruct((NUM_GRAPHS, D), jnp.float32),
)

def _pred_body(g_ref, pw_ref, pb_ref, out_ref):
    out_ref[0] = (jnp.dot(g_ref[...], pw_ref[0],
                          preferred_element_type=jnp.float32) + pb_ref[0])


_pred_call = pl.pallas_call(
    _pred_body,
    grid=(MAX_SEQ_LEN,),
    in_specs=[
        pl.BlockSpec((NUM_GRAPHS, D), lambda s: (0, 0)),
        pl.BlockSpec((1, D, NUM_VOCAB), lambda s: (s, 0, 0)),
        pl.BlockSpec((1, 1, NUM_VOCAB), lambda s: (s, 0, 0)),
    ],
    out_specs=pl.BlockSpec((1, NUM_GRAPHS, NUM_VOCAB), lambda s: (s, 0, 0)),
    out_shape=jax.ShapeDtypeStruct((MAX_SEQ_LEN, NUM_GRAPHS, NUM_VOCAB),
                                   jnp.float32),
)


def kernel(x, edge_index, node_depth, batch, edge_attr, params):
    x = x.astype(jnp.int32)
    src = edge_index[0].astype(jnp.int32)
    dst = edge_index[1].astype(jnp.int32)
    et = edge_attr.astype(jnp.int32)

    gidx_p = jnp.concatenate(
        [et * N + src, jnp.zeros((EPAD - E,), jnp.int32)])
    dst_p = jnp.concatenate(
        [dst, jnp.full((EPAD - E,), N, jnp.int32)])
    zeros_hbm = jnp.zeros((ZR, D), jnp.float32)

    xpad = jnp.concatenate(
        [x, node_depth.reshape(-1, 1).astype(jnp.int32),
         jnp.zeros((N, 125), jnp.int32)], axis=1)
    ae_pad = jnp.concatenate(
        [params['attr_emb'],
         jnp.zeros((1024 - NUM_NODE_ATTRS, D), jnp.float32)], axis=0)
    h0 = _enc_call(xpad, params['type_emb'], ae_pad, params['depth_emb'])

    states = [h0]
    h = h0
    for l, T in enumerate(LAYER_TIMESTEPS):
        res_list = [states[i] for i in RESIDUALS_MAP.get(l, [])]
        nres = len(res_list)
        wih = params['gru_wih_%d' % l]
        wiaT = wih[:, :D].T
        whhT = params['gru_whh_%d' % l].T
        bih = params['gru_bih_%d' % l].reshape(1, 3 * D)
        bhh = params['gru_bhh_%d' % l].reshape(1, 3 * D)
        res_args = []
        for j, rs in enumerate(res_list):
            res_args.append(rs)
            res_args.append(wih[:, D * (j + 1):D * (j + 2)].T)
        W = params['edge_w_%d' % l]
        for _ in range(T):
            hr = _hr_call(h, W)
            aggp = _edge_pass(hr.reshape(NR * N, D), gidx_p, dst_p,
                              zeros_hbm)
            h = _gru_calls[nres](aggp, h, wiaT, whhT, bih, bhh, *res_args)
        states.append(h)

    batch3 = batch.astype(jnp.int32).reshape(NBLKE, 1, RBE)
    g = _cls_call(h, h0,
                  params['cl_w'][:, :D].T, params['cl_w'][:, D:].T,
                  params['cr_w'][:, :D].T, params['cr_w'][:, D:].T,
                  params['cl_b'].reshape(1, D), params['cr_b'].reshape(1, D),
                  batch3)
    pwT = params['pred_w'].transpose(0, 2, 1)
    pb3 = params['pred_b'].reshape(MAX_SEQ_LEN, 1, NUM_VOCAB)
    return _pred_call(g, pwT, pb3)
